# baseline (speedup 1.0000x reference)
import functools

import jax
import jax.numpy as jnp
from jax import lax
from jax.experimental import pallas as pl
from jax.experimental.pallas import tpu as pltpu
from jax.experimental.pallas import tpu_sc as plsc

N = 10000
NPAD = 10240
N_OCC = 3
NK = 19
NKP = 20
NP = 6
NF = 44
FP = 64
FPI = FP // 2
SF = 25
SFP = 32
Q = NPAD * NP
EPS = 1e-5
SHIFT = 0.6931

NWORK = 32
PW = Q // NWORK
QPAD = Q
B = 40
NITER = PW // B
BR = B * NK

TN = 640
NTILES = NPAD // TN
TNR = TN * FPI // 128
TM = 1280
MTILES = NPAD // TM


def _sc_body(table_hbm, idx_hbm, out_hbm, idx_v, rows_v, x_v, g0, g1, o0, o1):
    wid = lax.axis_index("s") * 2 + lax.axis_index("c")
    base = wid * PW
    pltpu.sync_copy(idx_hbm.at[pl.ds(base * NK, PW * NK)], idx_v)

    def gather_start(it, buf, sem):
        pltpu.make_async_copy(
            table_hbm.at[idx_v.at[pl.ds(it * BR, BR)]],
            rows_v.at[buf], sem).start()

    def gather_wait(buf, sem):
        pltpu.make_async_copy(
            table_hbm.at[idx_v.at[pl.ds(0, BR)]],
            rows_v.at[buf], sem).wait()

    def out_start(it, buf, sem):
        pltpu.make_async_copy(
            x_v.at[buf], out_hbm.at[pl.ds(base + it * B, B)], sem).start()

    def out_wait(buf, sem):
        pltpu.make_async_copy(
            x_v.at[buf], out_hbm.at[pl.ds(base, B)], sem).wait()

    def accumulate(buf):
        @pl.loop(0, B, step=2)
        def _(j):
            for u in range(2):
                r0 = (j + u) * NK
                for c in range(FPI // 16):
                    sl = pl.ds(c * 16, 16)
                    bf = jnp.bfloat16
                    a = plsc.bitcast(rows_v[buf, r0, sl], bf)
                    b = plsc.bitcast(rows_v[buf, r0 + 1, sl], bf)
                    for r in range(2, NK, 2):
                        a = a + plsc.bitcast(rows_v[buf, r0 + r, sl], bf)
                    for r in range(3, NK, 2):
                        b = b + plsc.bitcast(rows_v[buf, r0 + r, sl], bf)
                    x_v[buf, j + u, sl] = plsc.bitcast(a + b, jnp.int32)

    gather_start(0, 0, g0)

    @pl.loop(0, (NITER - 1) // 2)
    def _(t):
        it0 = 2 * t
        gather_start(it0 + 1, 1, g1)
        gather_wait(0, g0)

        @pl.when(t > 0)
        def _():
            out_wait(0, o0)

        accumulate(0)
        out_start(it0, 0, o0)

        it1 = it0 + 1
        gather_start(it1 + 1, 0, g0)
        gather_wait(1, g1)

        @pl.when(t > 0)
        def _():
            out_wait(1, o1)

        accumulate(1)
        out_start(it1, 1, o1)

    gather_start(NITER - 1, 1, g1)
    gather_wait(0, g0)
    out_wait(0, o0)
    accumulate(0)
    out_start(NITER - 2, 0, o0)
    gather_wait(1, g1)
    out_wait(1, o1)
    accumulate(1)
    out_start(NITER - 1, 1, o1)
    out_wait(0, o0)
    out_wait(1, o1)


def _gather_sum(table_i32, idx_flat):
    mesh = plsc.VectorSubcoreMesh(core_axis_name="c", subcore_axis_name="s")
    kfn = pl.kernel(
        _sc_body,
        out_type=jax.ShapeDtypeStruct((QPAD, FPI), jnp.int32),
        mesh=mesh,
        compiler_params=pltpu.CompilerParams(use_tc_tiling_on_sc=False,
                                             needs_layout_passes=False),
        scratch_types=[
            pltpu.VMEM((PW * NK,), jnp.int32),
            pltpu.VMEM((2, BR, FPI), jnp.int32),
            pltpu.VMEM((2, B, FPI), jnp.int32),
            pltpu.SemaphoreType.DMA,
            pltpu.SemaphoreType.DMA,
            pltpu.SemaphoreType.DMA,
            pltpu.SemaphoreType.DMA,
        ],
    )
    return kfn(table_i32, idx_flat)


def _rne_bf16_bits(u):
    ui = jax.lax.bitcast_convert_type(u, jnp.uint32)
    r = ui + jnp.uint32(0x7FFF) + ((ui >> 16) & jnp.uint32(1))
    return r >> 16


def _table_body(h_ref, we_ref, wo_ref, o_ref):
    ge = jnp.dot(h_ref[...], we_ref[...], preferred_element_type=jnp.float32)
    go = jnp.dot(h_ref[...], wo_ref[...], preferred_element_type=jnp.float32)
    packed = _rne_bf16_bits(ge) | (_rne_bf16_bits(go) << 16)
    packed = jax.lax.bitcast_convert_type(packed, jnp.int32)
    o_ref[...] = packed.reshape(TM * NKP * FPI // 128, 128)


def _table_call(h, we, wo):
    fin = we.shape[0]
    rows_per_tile = TM * NKP * FPI // 128
    return pl.pallas_call(
        _table_body,
        grid=(MTILES,),
        in_specs=[pl.BlockSpec((TM, fin), lambda i: (i, 0)),
                  pl.BlockSpec((fin, NKP * FPI), lambda i: (0, 0)),
                  pl.BlockSpec((fin, NKP * FPI), lambda i: (0, 0))],
        out_specs=pl.BlockSpec((rows_per_tile, 128), lambda i: (i, 0)),
        out_shape=jax.ShapeDtypeStruct((MTILES * rows_per_tile, 128),
                                       jnp.int32),
    )(h, we, wo)


def _unpack_x(ref):
    return ref[...].astype(jnp.float32)


def _bn_sum(xrefs, b_ref, g_ref, be_ref):
    b = b_ref[0:1, :]
    xs = [_unpack_x(r) + b for r in xrefs]
    m = xs[0]
    for xi in xs[1:]:
        m = m + xi
    m = m * (1.0 / NP)
    var = (xs[0] - m) ** 2
    for xi in xs[1:]:
        var = var + (xi - m) ** 2
    var = var * (1.0 / NP)
    inv = lax.rsqrt(var + EPS)
    g = g_ref[0:1, :]
    be = be_ref[0:1, :]
    h = (xs[0] - m) * inv * g + be
    for xi in xs[1:]:
        h = h + (xi - m) * inv * g + be
    return h


def _bn_body(x0, x1, x2, x3, x4, x5, b_ref, g_ref, be_ref, o_ref):
    h = _bn_sum((x0, x1, x2, x3, x4, x5), b_ref, g_ref, be_ref)
    o_ref[...] = h.astype(jnp.bfloat16)


def _head_body(x0, x1, x2, x3, x4, x5, b_ref, g_ref, be_ref,
               wc_ref, bc_ref, gc_ref, bec_ref, wl_ref, bl_ref, wfb_ref,
               o_ref, acc_ref):
    i = pl.program_id(0)

    @pl.when(i == 0)
    def _():
        acc_ref[...] = jnp.zeros_like(acc_ref)

    h = _bn_sum((x0, x1, x2, x3, x4, x5), b_ref, g_ref, be_ref)
    hc = jnp.dot(h, wc_ref[...], preferred_element_type=jnp.float32)
    hc = hc + bc_ref[0:1, :]
    lane = lax.broadcasted_iota(jnp.int32, hc.shape, 1)
    mask = lane < SF
    mu = jnp.sum(hc, axis=-1, keepdims=True) * (1.0 / SF)
    d = jnp.where(mask, hc - mu, 0.0)
    sig = jnp.sum(d * d, axis=-1, keepdims=True) * (1.0 / SF)
    hn = d * lax.rsqrt(sig + EPS) * gc_ref[0:1, :] + bec_ref[0:1, :]
    sp = jnp.maximum(hn, 0.0) + jnp.log(1.0 + jnp.exp(-jnp.abs(hn))) - SHIFT
    sp = jnp.where(mask, sp, 0.0)
    hl = jnp.dot(sp, wl_ref[...], preferred_element_type=jnp.float32)
    hl = hl + bl_ref[0:1, :]
    rowi = lax.broadcasted_iota(jnp.int32, hl.shape, 0) + i * TN
    hl = jnp.where(rowi < N, hl, 0.0)
    acc_ref[0:1, 0:SFP] += jnp.sum(hl, axis=0, keepdims=True)

    @pl.when(i == NTILES - 1)
    def _():
        gmean = acc_ref[0:1, 0:SFP] * (1.0 / N)
        val = jnp.sum(gmean * wfb_ref[0:1, :]) + wfb_ref[1, 0]
        o_ref[...] = jnp.full((8, 128), val, jnp.float32)


def _x_specs():
    return [pl.BlockSpec((TN, FP), functools.partial(
        lambda p, i: (p * NTILES + i, 0), p)) for p in range(NP)]


def _vec_spec():
    return pl.BlockSpec((8, FP), lambda i: (0, 0))


def _pad_row(v, width):
    out = jnp.zeros((8, width), jnp.float32)
    return out.at[0, : v.shape[0]].set(v)


def kernel(x, edge_index, W1, b1, g1, be1, W2, b2, g2, be2,
           Wc, bc, gc, bec, Wl, bl, Wf, bf):
    src = edge_index[0].astype(jnp.int32)
    src3 = src.reshape(N, NP, NK)
    idx3 = src3 * NKP + jnp.arange(NK, dtype=jnp.int32)[None, None, :]
    idxp = jnp.transpose(idx3, (1, 0, 2))
    idxp = jnp.pad(idxp, ((0, 0), (0, NPAD - N), (0, 0)))
    idx_flat = idxp.reshape(-1)

    def wconv(W, F, fin):
        Wr = W.reshape(NK, F, NF)
        Wr = jnp.pad(Wr, ((0, NKP - NK), (0, fin - F), (0, FP - NF)))
        Wb = Wr.transpose(1, 0, 2).reshape(fin, NKP * FP).astype(jnp.bfloat16)
        return Wb[:, 0::2], Wb[:, 1::2]

    W1e, W1o = wconv(W1, N_OCC, 8)
    xp = jnp.pad(x, ((0, NPAD - N), (0, 8 - N_OCC))).astype(jnp.bfloat16)
    W2e, W2o = wconv(W2, NF, FP)

    b1p, g1p, be1p = _pad_row(b1, FP), _pad_row(g1, FP), _pad_row(be1, FP)
    b2p, g2p, be2p = _pad_row(b2, FP), _pad_row(g2, FP), _pad_row(be2, FP)
    Wcp = jnp.pad(Wc, ((0, FP - NF), (0, SFP - SF)))
    bcp, gcp, becp = _pad_row(bc, SFP), _pad_row(gc, SFP), _pad_row(bec, SFP)
    Wlp = jnp.pad(Wl, ((0, SFP - SF), (0, SFP - SF)))
    blp = _pad_row(bl, SFP)
    wfb = jnp.zeros((8, SFP), jnp.float32)
    wfb = wfb.at[0, :SF].set(Wf[:, 0]).at[1, 0].set(bf[0])

    T1 = _table_call(xp, W1e, W1o)

    X1 = _gather_sum(T1.reshape(NPAD * NKP, FPI), idx_flat)
    X1t = jax.lax.bitcast_convert_type(X1, jnp.bfloat16).reshape(QPAD, FP)

    h1 = pl.pallas_call(
        _bn_body,
        grid=(NTILES,),
        in_specs=_x_specs() + [_vec_spec()] * 3,
        out_specs=pl.BlockSpec((TN, FP), lambda i: (i, 0)),
        out_shape=jax.ShapeDtypeStruct((NPAD, FP), jnp.bfloat16),
    )(X1t, X1t, X1t, X1t, X1t, X1t, b1p, g1p, be1p)

    T2 = _table_call(h1, W2e, W2o)

    X2 = _gather_sum(T2.reshape(NPAD * NKP, FPI), idx_flat)
    X2t = jax.lax.bitcast_convert_type(X2, jnp.bfloat16).reshape(QPAD, FP)

    sfv = pl.BlockSpec((8, SFP), lambda i: (0, 0))
    out = pl.pallas_call(
        _head_body,
        grid=(NTILES,),
        in_specs=_x_specs() + [_vec_spec()] * 3
        + [pl.BlockSpec((FP, SFP), lambda i: (0, 0)), sfv, sfv, sfv,
           pl.BlockSpec((SFP, SFP), lambda i: (0, 0)), sfv, sfv],
        out_specs=pl.BlockSpec((8, 128), lambda i: (0, 0)),
        out_shape=jax.ShapeDtypeStruct((8, 128), jnp.float32),
        scratch_shapes=[pltpu.VMEM((8, 128), jnp.float32)],
    )(X2t, X2t, X2t, X2t, X2t, X2t, b2p, g2p, be2p,
      Wcp, bcp, gcp, becp, Wlp, blp, wfb)

    return out[0:1, 0]


# speedup vs baseline: 1.4162x; 1.4162x over previous
import functools

import jax
import jax.numpy as jnp
from jax import lax
from jax.experimental import pallas as pl
from jax.experimental.pallas import tpu as pltpu
from jax.experimental.pallas import tpu_sc as plsc

N = 10000
N_OCC = 3
NK = 19
NP = 6
NF = 44
FP = 64
SF = 25
SFP = 32
Q = N * NP
EPS = 1e-5
SHIFT = 0.6931

NWORK = 32
PW = 1880
QPAD = NWORK * PW
B = 40
NITER = PW // B
BR = B * NK

TN = 400
NTILES = N // TN


def _sc_body(table_hbm, idx_hbm, out_hbm, idx_v, rows_v, x_v, g0, g1, o0, o1):
    wid = lax.axis_index("s") * 2 + lax.axis_index("c")
    base = wid * PW
    pltpu.sync_copy(idx_hbm.at[pl.ds(base * NK, PW * NK)], idx_v)

    def gather_start(it, buf, sem):
        pltpu.make_async_copy(
            table_hbm.at[idx_v.at[pl.ds(it * BR, BR)]],
            rows_v.at[buf], sem).start()

    def gather_wait(buf, sem):
        pltpu.make_async_copy(
            table_hbm.at[idx_v.at[pl.ds(0, BR)]],
            rows_v.at[buf], sem).wait()

    def out_start(it, buf, sem):
        pltpu.make_async_copy(
            x_v.at[buf], out_hbm.at[pl.ds(base + it * B, B)], sem).start()

    def out_wait(buf, sem):
        pltpu.make_async_copy(
            x_v.at[buf], out_hbm.at[pl.ds(base, B)], sem).wait()

    def accumulate(buf):
        @pl.loop(0, B, step=2)
        def _(j):
            for u in range(2):
                r0 = (j + u) * NK
                for c in range(FP // 32):
                    sl = pl.ds(c * 32, 32)
                    a = rows_v[buf, r0, sl]
                    b = rows_v[buf, r0 + 1, sl]
                    for r in range(2, NK, 2):
                        a = a + rows_v[buf, r0 + r, sl]
                    for r in range(3, NK, 2):
                        b = b + rows_v[buf, r0 + r, sl]
                    x_v[buf, j + u, sl] = a + b

    gather_start(0, 0, g0)

    @pl.loop(0, (NITER - 1) // 2)
    def _(t):
        it0 = 2 * t
        gather_start(it0 + 1, 1, g1)
        gather_wait(0, g0)

        @pl.when(t > 0)
        def _():
            out_wait(0, o0)

        accumulate(0)
        out_start(it0, 0, o0)

        it1 = it0 + 1
        gather_start(it1 + 1, 0, g0)
        gather_wait(1, g1)

        @pl.when(t > 0)
        def _():
            out_wait(1, o1)

        accumulate(1)
        out_start(it1, 1, o1)

    gather_wait(0, g0)
    out_wait(0, o0)
    accumulate(0)
    out_start(NITER - 1, 0, o0)
    out_wait(0, o0)
    out_wait(1, o1)


def _gather_sum(table, idx_flat):
    mesh = plsc.VectorSubcoreMesh(core_axis_name="c", subcore_axis_name="s")
    kfn = pl.kernel(
        _sc_body,
        out_type=jax.ShapeDtypeStruct((QPAD, FP), jnp.bfloat16),
        mesh=mesh,
        compiler_params=pltpu.CompilerParams(use_tc_tiling_on_sc=False),
        scratch_types=[
            pltpu.VMEM((PW * NK,), jnp.int32),
            pltpu.VMEM((2, BR, FP), jnp.bfloat16),
            pltpu.VMEM((2, B, FP), jnp.bfloat16),
            pltpu.SemaphoreType.DMA,
            pltpu.SemaphoreType.DMA,
            pltpu.SemaphoreType.DMA,
            pltpu.SemaphoreType.DMA,
        ],
    )
    return kfn(table, idx_flat)


def _table_body(h_ref, w_ref, o_ref):
    o_ref[...] = jnp.dot(h_ref[...], w_ref[0],
                         preferred_element_type=jnp.float32
                         ).astype(jnp.bfloat16)


def _table_call(h, w3):
    fin = w3.shape[1]
    return pl.pallas_call(
        _table_body,
        grid=(NK,),
        in_specs=[pl.BlockSpec((N, fin), lambda k: (0, 0)),
                  pl.BlockSpec((1, fin, FP), lambda k: (k, 0, 0))],
        out_specs=pl.BlockSpec((N, FP), lambda k: (k, 0)),
        out_shape=jax.ShapeDtypeStruct((NK * N, FP), jnp.bfloat16),
    )(h, w3)


def _bn_sum(xrefs, b_ref, g_ref, be_ref):
    b = b_ref[0:1, :]
    xs = [r[...].astype(jnp.float32) + b for r in xrefs]
    m = xs[0]
    for xi in xs[1:]:
        m = m + xi
    m = m * (1.0 / NP)
    var = (xs[0] - m) ** 2
    for xi in xs[1:]:
        var = var + (xi - m) ** 2
    var = var * (1.0 / NP)
    inv = lax.rsqrt(var + EPS)
    g = g_ref[0:1, :]
    be = be_ref[0:1, :]
    h = (xs[0] - m) * inv * g + be
    for xi in xs[1:]:
        h = h + (xi - m) * inv * g + be
    return h


def _bn_body(x0, x1, x2, x3, x4, x5, b_ref, g_ref, be_ref, o_ref):
    h = _bn_sum((x0, x1, x2, x3, x4, x5), b_ref, g_ref, be_ref)
    o_ref[...] = h.astype(jnp.bfloat16)


def _head_body(x0, x1, x2, x3, x4, x5, b_ref, g_ref, be_ref,
               wc_ref, bc_ref, gc_ref, bec_ref, wl_ref, bl_ref, wfb_ref,
               o_ref, acc_ref):
    i = pl.program_id(0)

    @pl.when(i == 0)
    def _():
        acc_ref[...] = jnp.zeros_like(acc_ref)

    h = _bn_sum((x0, x1, x2, x3, x4, x5), b_ref, g_ref, be_ref)
    hc = jnp.dot(h, wc_ref[...], preferred_element_type=jnp.float32)
    hc = hc + bc_ref[0:1, :]
    lane = lax.broadcasted_iota(jnp.int32, hc.shape, 1)
    mask = lane < SF
    mu = jnp.sum(hc, axis=-1, keepdims=True) * (1.0 / SF)
    d = jnp.where(mask, hc - mu, 0.0)
    sig = jnp.sum(d * d, axis=-1, keepdims=True) * (1.0 / SF)
    hn = d * lax.rsqrt(sig + EPS) * gc_ref[0:1, :] + bec_ref[0:1, :]
    sp = jnp.maximum(hn, 0.0) + jnp.log(1.0 + jnp.exp(-jnp.abs(hn))) - SHIFT
    sp = jnp.where(mask, sp, 0.0)
    hl = jnp.dot(sp, wl_ref[...], preferred_element_type=jnp.float32)
    hl = hl + bl_ref[0:1, :]
    acc_ref[0:1, 0:SFP] += jnp.sum(hl, axis=0, keepdims=True)

    @pl.when(i == NTILES - 1)
    def _():
        gmean = acc_ref[0:1, 0:SFP] * (1.0 / N)
        val = jnp.sum(gmean * wfb_ref[0:1, :]) + wfb_ref[1, 0]
        o_ref[...] = jnp.full((8, 128), val, jnp.float32)


def _x_specs():
    return [pl.BlockSpec((TN, FP), functools.partial(
        lambda p, i: (p * NTILES + i, 0), p)) for p in range(NP)]


def _vec_spec():
    return pl.BlockSpec((8, FP), lambda i: (0, 0))


def _pad_row(v, width):
    out = jnp.zeros((8, width), jnp.float32)
    return out.at[0, : v.shape[0]].set(v)


def kernel(x, edge_index, W1, b1, g1, be1, W2, b2, g2, be2,
           Wc, bc, gc, bec, Wl, bl, Wf, bf):
    src = edge_index[0].astype(jnp.int32)
    src3 = src.reshape(N, NP, NK)
    idx3 = src3 + (jnp.arange(NK, dtype=jnp.int32) * N)[None, None, :]
    idxp = jnp.transpose(idx3, (1, 0, 2)).reshape(Q, NK)
    idxp = jnp.concatenate(
        [idxp, jnp.zeros((QPAD - Q, NK), jnp.int32)], axis=0)
    idx_flat = idxp.reshape(-1)

    def wconv(W, F, fin):
        Wr = W.reshape(NK, F, NF)
        Wr = jnp.pad(Wr, ((0, 0), (0, fin - F), (0, FP - NF)))
        return Wr.astype(jnp.bfloat16)

    W1r = wconv(W1, N_OCC, 8)
    xp = jnp.pad(x, ((0, 0), (0, 8 - N_OCC))).astype(jnp.bfloat16)
    W2r = wconv(W2, NF, FP)

    b1p, g1p, be1p = _pad_row(b1, FP), _pad_row(g1, FP), _pad_row(be1, FP)
    b2p, g2p, be2p = _pad_row(b2, FP), _pad_row(g2, FP), _pad_row(be2, FP)
    Wcp = jnp.pad(Wc, ((0, FP - NF), (0, SFP - SF)))
    bcp, gcp, becp = _pad_row(bc, SFP), _pad_row(gc, SFP), _pad_row(bec, SFP)
    Wlp = jnp.pad(Wl, ((0, SFP - SF), (0, SFP - SF)))
    blp = _pad_row(bl, SFP)
    wfb = jnp.zeros((8, SFP), jnp.float32)
    wfb = wfb.at[0, :SF].set(Wf[:, 0]).at[1, 0].set(bf[0])

    T1 = _table_call(xp, W1r)

    X1 = _gather_sum(T1, idx_flat)

    h1 = pl.pallas_call(
        _bn_body,
        grid=(NTILES,),
        in_specs=_x_specs() + [_vec_spec()] * 3,
        out_specs=pl.BlockSpec((TN, FP), lambda i: (i, 0)),
        out_shape=jax.ShapeDtypeStruct((N, FP), jnp.bfloat16),
    )(X1, X1, X1, X1, X1, X1, b1p, g1p, be1p)

    T2 = _table_call(h1, W2r)

    X2 = _gather_sum(T2, idx_flat)

    sfv = pl.BlockSpec((8, SFP), lambda i: (0, 0))
    out = pl.pallas_call(
        _head_body,
        grid=(NTILES,),
        in_specs=_x_specs() + [_vec_spec()] * 3
        + [pl.BlockSpec((FP, SFP), lambda i: (0, 0)), sfv, sfv, sfv,
           pl.BlockSpec((SFP, SFP), lambda i: (0, 0)), sfv, sfv],
        out_specs=pl.BlockSpec((8, 128), lambda i: (0, 0)),
        out_shape=jax.ShapeDtypeStruct((8, 128), jnp.float32),
        scratch_shapes=[pltpu.VMEM((8, 128), jnp.float32)],
    )(X2, X2, X2, X2, X2, X2, b2p, g2p, be2p,
      Wcp, bcp, gcp, becp, Wlp, blp, wfb)

    return out[0:1, 0]
